# baseline (speedup 1.0000x reference)
import functools

import jax
import jax.numpy as jnp
from jax import lax
from jax.experimental import pallas as pl
from jax.experimental.pallas import tpu as pltpu
from jax.experimental.pallas import tpu_sc as plsc

_SAMPLE = 4000
_K = 8


def _sc_gather(tables, idx):
    E = idx.shape[0]
    nt = len(tables)
    info = plsc.get_sparse_core_info()
    NC, NS = info.num_cores, info.num_subcores
    NW = NC * NS
    per = E // NW
    mesh = plsc.VectorSubcoreMesh(core_axis_name="c", subcore_axis_name="s")

    @functools.partial(
        pl.kernel,
        mesh=mesh,
        out_type=tuple(
            jax.ShapeDtypeStruct((E,), jnp.float32) for _ in range(nt)
        ),
        scratch_types=[pltpu.VMEM((per,), jnp.int32)]
        + [pltpu.VMEM((per,), jnp.float32) for _ in range(nt)]
        + [pltpu.SemaphoreType.DMA for _ in range(nt)],
    )
    def k(*refs):
        t_hbm = refs[:nt]
        idx_hbm = refs[nt]
        o_hbm = refs[nt + 1 : 2 * nt + 1]
        idx_v = refs[2 * nt + 1]
        t_v = refs[2 * nt + 2 : 3 * nt + 2]
        sems = refs[3 * nt + 2 :]
        wid = lax.axis_index("s") * NC + lax.axis_index("c")
        base = wid * per
        pltpu.sync_copy(idx_hbm.at[pl.ds(base, per)], idx_v)
        cps = [
            pltpu.async_copy(t_hbm[j].at[idx_v], t_v[j], sems[j])
            for j in range(nt)
        ]
        for cp in cps:
            cp.wait()
        for j in range(nt):
            pltpu.sync_copy(t_v[j], o_hbm[j].at[pl.ds(base, per)])

    return k(*tables, idx)


def _knn_loss_body(rm_ref, rs_ref, cmt_ref, cst_ref, out_ref):
    rm = rm_ref[...]
    rs = rs_ref[...]
    sqm = None
    sqs = None
    for c in range(3):
        dm = rm[:, c : c + 1] - cmt_ref[c : c + 1, :]
        dsh = rs[:, c : c + 1] - cst_ref[c : c + 1, :]
        sqm = dm * dm if sqm is None else sqm + dm * dm
        sqs = dsh * dsh if sqs is None else sqs + dsh * dsh
    colid = lax.broadcasted_iota(jnp.int32, sqm.shape, 1)
    d = sqm
    acc = jnp.float32(0.0)
    big_i = jnp.int32(2**30)
    for k in range(_K + 1):
        m = jnp.min(d, axis=1, keepdims=True)
        cand = jnp.where(d == m, colid, big_i)
        jmin = jnp.min(cand, axis=1, keepdims=True)
        sel = colid == jmin
        if k > 0:
            ssel = jnp.sum(
                jnp.where(sel, sqs, jnp.float32(0.0)), axis=1, keepdims=True
            )
            acc = acc + jnp.sum(jnp.exp(-jnp.sqrt(m)) * ssel)
        d = jnp.where(sel, jnp.float32(jnp.inf), d)

    z = jnp.zeros((8, 128), jnp.float32)
    r0 = lax.broadcasted_iota(jnp.int32, (8, 128), 0)
    c0 = lax.broadcasted_iota(jnp.int32, (8, 128), 1)
    out_ref[...] = jnp.where((r0 == 0) & (c0 == 0), acc, z)


def kernel(means, sh0):
    if sh0.ndim == 2:
        sh0 = sh0[:, None, :]
    num = means.shape[0]
    n = min(_SAMPLE, num)
    idx = jax.random.randint(jax.random.key(42), (n,), 0, num)
    E = ((n + 255) // 256) * 256
    idx_p = jnp.concatenate(
        [idx.astype(jnp.int32), jnp.zeros((E - n,), jnp.int32)]
    )
    cols = [means[:, c] + 0.0 * means[:, c] for c in range(3)] + [
        sh0[:, 0, c] + 0.0 * sh0[:, 0, c] for c in range(3)
    ]
    gathered = _sc_gather(cols, idx_p)
    smt = jnp.stack(gathered[:3], axis=0)[:, :n]
    sst = jnp.stack(gathered[3:], axis=0)[:, :n]
    sm = smt.T
    ss = sst.T

    rb = 400 if n % 400 == 0 else n
    grid = n // rb
    out = pl.pallas_call(
        _knn_loss_body,
        grid=(grid,),
        in_specs=[
            pl.BlockSpec((rb, 3), lambda i: (i, 0)),
            pl.BlockSpec((rb, 3), lambda i: (i, 0)),
            pl.BlockSpec((3, n), lambda i: (0, 0)),
            pl.BlockSpec((3, n), lambda i: (0, 0)),
        ],
        out_specs=pl.BlockSpec((8, 128), lambda i: (i, 0)),
        out_shape=jax.ShapeDtypeStruct((grid * 8, 128), jnp.float32),
        compiler_params=pltpu.CompilerParams(
            dimension_semantics=("parallel",)
        ),
    )(sm, ss, smt, sst)
    return jnp.sum(out) / jnp.float32(n * _K * 3)


# speedup vs baseline: 1.0023x; 1.0023x over previous
import functools

import jax
import jax.numpy as jnp
from jax import lax
from jax.experimental import pallas as pl
from jax.experimental.pallas import tpu as pltpu
from jax.experimental.pallas import tpu_sc as plsc

_SAMPLE = 4000
_K = 8


def _sc_gather(tables, idx):
    E = idx.shape[0]
    nt = len(tables)
    info = plsc.get_sparse_core_info()
    NC, NS = info.num_cores, info.num_subcores
    NW = NC * NS
    per = E // NW
    mesh = plsc.VectorSubcoreMesh(core_axis_name="c", subcore_axis_name="s")

    @functools.partial(
        pl.kernel,
        mesh=mesh,
        out_type=tuple(
            jax.ShapeDtypeStruct((E,), jnp.float32) for _ in range(nt)
        ),
        scratch_types=[pltpu.VMEM((per,), jnp.int32)]
        + [pltpu.VMEM((per,), jnp.float32) for _ in range(nt)]
        + [pltpu.SemaphoreType.DMA for _ in range(nt)],
    )
    def k(*refs):
        t_hbm = refs[:nt]
        idx_hbm = refs[nt]
        o_hbm = refs[nt + 1 : 2 * nt + 1]
        idx_v = refs[2 * nt + 1]
        t_v = refs[2 * nt + 2 : 3 * nt + 2]
        sems = refs[3 * nt + 2 :]
        wid = lax.axis_index("s") * NC + lax.axis_index("c")
        base = wid * per
        pltpu.sync_copy(idx_hbm.at[pl.ds(base, per)], idx_v)
        cps = [
            pltpu.async_copy(t_hbm[j].at[idx_v], t_v[j], sems[j])
            for j in range(nt)
        ]
        for cp in cps:
            cp.wait()
        for j in range(nt):
            pltpu.sync_copy(t_v[j], o_hbm[j].at[pl.ds(base, per)])

    return k(*tables, idx)


def _knn_loss_body(rm_ref, rs_ref, cmt_ref, cst_ref, out_ref):
    i = pl.program_id(0)
    rm = rm_ref[...]
    rs = rs_ref[...]
    sqm = None
    sqs = None
    for c in range(3):
        dm = rm[:, c : c + 1] - cmt_ref[c : c + 1, :]
        dsh = rs[:, c : c + 1] - cst_ref[c : c + 1, :]
        sqm = dm * dm if sqm is None else sqm + dm * dm
        sqs = dsh * dsh if sqs is None else sqs + dsh * dsh
    colid = lax.broadcasted_iota(jnp.int32, sqm.shape, 1)
    d = sqm
    acc = jnp.float32(0.0)
    big_i = jnp.int32(2**30)
    for k in range(_K + 1):
        m = jnp.min(d, axis=1, keepdims=True)
        cand = jnp.where(d == m, colid, big_i)
        jmin = jnp.min(cand, axis=1, keepdims=True)
        sel = colid == jmin
        if k > 0:
            ssel = jnp.sum(
                jnp.where(sel, sqs, jnp.float32(0.0)), axis=1, keepdims=True
            )
            acc = acc + jnp.sum(jnp.exp(-jnp.sqrt(m)) * ssel)
        d = jnp.where(sel, jnp.float32(jnp.inf), d)

    @pl.when(i == 0)
    def _():
        out_ref[0, 0] = acc

    @pl.when(i > 0)
    def _():
        out_ref[0, 0] = out_ref[0, 0] + acc


def kernel(means, sh0):
    if sh0.ndim == 2:
        sh0 = sh0[:, None, :]
    num = means.shape[0]
    n = min(_SAMPLE, num)
    idx = jax.random.randint(jax.random.key(42), (n,), 0, num)
    E = ((n + 255) // 256) * 256
    idx_p = jnp.concatenate(
        [idx.astype(jnp.int32), jnp.zeros((E - n,), jnp.int32)]
    )
    cols = [means[:, c] + 0.0 * means[:, c] for c in range(3)] + [
        sh0[:, 0, c] + 0.0 * sh0[:, 0, c] for c in range(3)
    ]
    gathered = _sc_gather(cols, idx_p)
    smt = jnp.stack(gathered[:3], axis=0)[:, :n]
    sst = jnp.stack(gathered[3:], axis=0)[:, :n]
    sm = smt.T
    ss = sst.T

    rb = 400 if n % 400 == 0 else n
    grid = n // rb
    out = pl.pallas_call(
        _knn_loss_body,
        grid=(grid,),
        in_specs=[
            pl.BlockSpec((rb, 3), lambda i: (i, 0)),
            pl.BlockSpec((rb, 3), lambda i: (i, 0)),
            pl.BlockSpec((3, n), lambda i: (0, 0)),
            pl.BlockSpec((3, n), lambda i: (0, 0)),
        ],
        out_specs=pl.BlockSpec(
            (1, 1), lambda i: (0, 0), memory_space=pltpu.SMEM
        ),
        out_shape=jax.ShapeDtypeStruct((1, 1), jnp.float32),
    )(sm, ss, smt, sst)
    return out[0, 0] / jnp.float32(n * _K * 3)
